# Initial kernel scaffold; baseline (speedup 1.0000x reference)
import jax
import jax.numpy as jnp
from jax.experimental import pallas as pl


def kernel(x, edge_index, batch, W1, as1, ad1, b1, W2, as2, ad2, b2, W3, as3, ad3, b3, W4, as4, ad4, b4, Wih, Whh, bih, bhh, Wout, bout):
    raise NotImplementedError("write your pallas kernel here")


# speedup vs baseline: 37.3463x; 37.3463x over previous
import functools
import jax
import jax.numpy as jnp
from jax import lax
from jax.experimental import pallas as pl
from jax.experimental.pallas import tpu as pltpu
from jax.experimental.pallas import tpu_sc as plsc

N = 10000
E = 160000
B = 16
NC = 2
NS = 16
f32 = jnp.float32


def _make_sc_edge(heads, outc, half, R):
    EPT = E // NS
    K = 80
    NBLK = EPT // K
    NCH = half // 16

    mesh = plsc.VectorSubcoreMesh(core_axis_name="c", subcore_axis_name="s")

    @functools.partial(
        pl.kernel,
        out_type=[jax.ShapeDtypeStruct((N, R), f32)] * 2,
        mesh=mesh,
        scratch_types=[
            pltpu.VMEM((K,), jnp.int32),
            pltpu.VMEM((K,), jnp.int32),
            pltpu.VMEM((K, R), f32),
            pltpu.VMEM((K, 16), f32),
            pltpu.VMEM_SHARED((N, R), f32),
            pltpu.SemaphoreType.DMA,
            pltpu.SemaphoreType.DMA,
        ],
        compiler_params=pltpu.CompilerParams(use_tc_tiling_on_sc=False),
    )
    def sc_edge(g0, g1, d, src_h, dst_h, zz, acc0_o, acc1_o,
                idx_s, idx_d, gbuf, dbuf, acc_sh, sem_g, sem_d):
        c = lax.axis_index("c")
        s = lax.axis_index("s")

        def tile_rows(do):
            @pl.when(s < NS - 1)
            def _():
                do(pl.multiple_of(s * 632, 8), 632)

            @pl.when(s == NS - 1)
            def _():
                do((NS - 1) * 632, N - (NS - 1) * 632)

        tile_rows(lambda r0, nr: pltpu.sync_copy(
            zz.at[pl.ds(r0, nr)], acc_sh.at[pl.ds(r0, nr)]))
        plsc.subcore_barrier()

        mask8 = lax.iota(jnp.int32, 16) < 8

        def run(g_hbm, hb):
            base0 = s * EPT

            def blk(b, carry):
                base = pl.multiple_of(base0 + b * K, 8)
                pltpu.sync_copy(src_h.at[pl.ds(base, K)], idx_s)
                pltpu.sync_copy(dst_h.at[pl.ds(base, K)], idx_d)
                cp_g = pltpu.async_copy(g_hbm.at[idx_s], gbuf, sem_g)
                cp_d = pltpu.async_copy(d.at[idx_d], dbuf, sem_d)
                cp_g.wait()
                cp_d.wait()

                def edge(e, cc):
                    a = gbuf[e, pl.ds(half, 16)]
                    ee = a + dbuf[e, :]
                    ee = jnp.where(ee > 0, ee, 0.2 * ee)
                    w = jnp.exp(ee)
                    gbuf[e, pl.ds(half, 16)] = w
                    for ch in range(NCH):
                        if outc >= 16:
                            wv = lax.broadcast(w[hb + (ch * 16) // outc], (16,))
                        else:
                            w_lo = lax.broadcast(w[hb + 2 * ch], (16,))
                            w_hi = lax.broadcast(w[hb + 2 * ch + 1], (16,))
                            wv = jnp.where(mask8, w_lo, w_hi)
                        gbuf[e, pl.ds(ch * 16, 16)] = gbuf[e, pl.ds(ch * 16, 16)] * wv
                    return cc

                lax.fori_loop(0, K, edge, 0)
                pltpu.sync_copy(gbuf, acc_sh.at[idx_d], add=True)
                return carry

            lax.fori_loop(0, NBLK, blk, 0)

        @pl.when(c == 0)
        def _():
            run(g0, 0)

        @pl.when(c == 1)
        def _():
            run(g1, heads // 2)

        plsc.subcore_barrier()

        @pl.when(c == 0)
        def _():
            tile_rows(lambda r0, nr: pltpu.sync_copy(
                acc_sh.at[pl.ds(r0, nr)], acc0_o.at[pl.ds(r0, nr)]))

        @pl.when(c == 1)
        def _():
            tile_rows(lambda r0, nr: pltpu.sync_copy(
                acc_sh.at[pl.ds(r0, nr)], acc1_o.at[pl.ds(r0, nr)]))

    return sc_edge


def _make_sc_edge4():
    EPT = E // (NS * NC)
    K = 40
    NBLK = EPT // K

    mesh = plsc.VectorSubcoreMesh(core_axis_name="c", subcore_axis_name="s")

    @functools.partial(
        pl.kernel,
        out_type=[jax.ShapeDtypeStruct((N, 16), f32)] * 2,
        mesh=mesh,
        scratch_types=[
            pltpu.VMEM((K,), jnp.int32),
            pltpu.VMEM((K,), jnp.int32),
            pltpu.VMEM((K, 16), f32),
            pltpu.VMEM((K, 16), f32),
            pltpu.VMEM_SHARED((N, 16), f32),
            pltpu.SemaphoreType.DMA,
            pltpu.SemaphoreType.DMA,
        ],
        compiler_params=pltpu.CompilerParams(use_tc_tiling_on_sc=False),
    )
    def sc_edge4(g, d, src_h, dst_h, zz, acc0_o, acc1_o,
                 idx_s, idx_d, gbuf, dbuf, acc_sh, sem_g, sem_d):
        c = lax.axis_index("c")
        s = lax.axis_index("s")

        def tile_rows(do):
            @pl.when(s < NS - 1)
            def _():
                do(pl.multiple_of(s * 632, 8), 632)

            @pl.when(s == NS - 1)
            def _():
                do((NS - 1) * 632, N - (NS - 1) * 632)

        tile_rows(lambda r0, nr: pltpu.sync_copy(
            zz.at[pl.ds(r0, nr)], acc_sh.at[pl.ds(r0, nr)]))
        plsc.subcore_barrier()

        mask4 = lax.iota(jnp.int32, 16) < 4
        wid = s * NC + c
        base0 = wid * EPT

        def blk(b, carry):
            base = pl.multiple_of(base0 + b * K, 8)
            pltpu.sync_copy(src_h.at[pl.ds(base, K)], idx_s)
            pltpu.sync_copy(dst_h.at[pl.ds(base, K)], idx_d)
            cp_g = pltpu.async_copy(g.at[idx_s], gbuf, sem_g)
            cp_d = pltpu.async_copy(d.at[idx_d], dbuf, sem_d)
            cp_g.wait()
            cp_d.wait()

        def edge(e, cc):
            a_row = gbuf[e, :]
            ev = a_row + dbuf[e, :]
            es = lax.broadcast(ev[4], (16,))
            es = jnp.where(es > 0, es, 0.2 * es)
            w = jnp.exp(es)
            gbuf[e, :] = jnp.where(mask4, a_row * w, w)
            return cc

        def blk2(b, carry):
            blk(b, carry)
            lax.fori_loop(0, K, edge, 0)
            pltpu.sync_copy(gbuf, acc_sh.at[idx_d], add=True)
            return carry

        lax.fori_loop(0, NBLK, blk2, 0)
        plsc.subcore_barrier()

        @pl.when(c == 0)
        def _():
            tile_rows(lambda r0, nr: pltpu.sync_copy(
                acc_sh.at[pl.ds(r0, nr)], acc0_o.at[pl.ds(r0, nr)]))

        @pl.when(c == 1)
        def _():
            tile_rows(lambda r0, nr: pltpu.sync_copy(
                acc_sh.at[pl.ds(r0, nr)], acc1_o.at[pl.ds(r0, nr)]))

    return sc_edge4


BLK = 2000
GRID = N // BLK


def _row_spec(ncol):
    return pl.BlockSpec((BLK, ncol), lambda i: (i, 0))


def _full_spec(shape):
    return pl.BlockSpec(shape, lambda i: (0,) * len(shape))


def _tc_first(x, m0, m1, md):
    def body(x_r, m0_r, m1_r, md_r, g0_r, g1_r, d_r):
        xx = x_r[...]
        g0_r[...] = jnp.dot(xx, m0_r[...], preferred_element_type=f32)
        g1_r[...] = jnp.dot(xx, m1_r[...], preferred_element_type=f32)
        d_r[...] = jnp.dot(xx, md_r[...], preferred_element_type=f32)

    R = m0.shape[1]
    return pl.pallas_call(
        body,
        grid=(GRID,),
        in_specs=[_row_spec(x.shape[1]), _full_spec(m0.shape),
                  _full_spec(m1.shape), _full_spec(md.shape)],
        out_specs=[_row_spec(R), _row_spec(R), _row_spec(16)],
        out_shape=[
            jax.ShapeDtypeStruct((N, R), f32),
            jax.ShapeDtypeStruct((N, R), f32),
            jax.ShapeDtypeStruct((N, 16), f32),
        ],
    )(x, m0, m1, md)


def _combine(acc0, acc1, g0p, g1p, dp, bprev, expm, half):
    H = jnp.concatenate([g0p[:, :half], g1p[:, :half]], axis=1)
    asrc = g0p[:, half:half + 8]
    adst = dp[:, :8]
    es = asrc + adst
    es = jnp.where(es > 0, es, 0.2 * es)
    ws = jnp.exp(es)
    denom = acc0[:, half:half + 8] + ws
    wsE = jnp.dot(ws, expm, preferred_element_type=f32)
    denE = jnp.dot(denom, expm, preferred_element_type=f32)
    accH = jnp.concatenate([acc0[:, :half], acc1[:, :half]], axis=1) + wsE * H
    h = accH / denE + bprev
    return jnp.where(h > 0, h, jnp.exp(h) - 1.0)


def _tc_boundary(acc0, acc1, g0p, g1p, dp, bprev, expm, m0, m1, md, half, two_out):
    def body(a0_r, a1_r, g0_r, g1_r, d_r, b_r, e_r, m0_r, m1_r, md_r, *outs):
        h = _combine(a0_r[...], a1_r[...], g0_r[...], g1_r[...], d_r[...],
                     b_r[...], e_r[...], half)
        outs[0][...] = jnp.dot(h, m0_r[...], preferred_element_type=f32)
        if two_out:
            outs[1][...] = jnp.dot(h, m1_r[...], preferred_element_type=f32)
            outs[2][...] = jnp.dot(h, md_r[...], preferred_element_type=f32)
        else:
            outs[1][...] = jnp.dot(h, md_r[...], preferred_element_type=f32)

    R = m0.shape[1]
    if two_out:
        out_shape = [
            jax.ShapeDtypeStruct((N, R), f32),
            jax.ShapeDtypeStruct((N, R), f32),
            jax.ShapeDtypeStruct((N, 16), f32),
        ]
        out_specs = [_row_spec(R), _row_spec(R), _row_spec(16)]
    else:
        out_shape = [
            jax.ShapeDtypeStruct((N, R), f32),
            jax.ShapeDtypeStruct((N, 16), f32),
        ]
        out_specs = [_row_spec(R), _row_spec(16)]
    Rp = acc0.shape[1]
    return pl.pallas_call(
        body,
        grid=(GRID,),
        in_specs=[_row_spec(Rp), _row_spec(Rp), _row_spec(Rp), _row_spec(Rp),
                  _row_spec(16), _full_spec(bprev.shape), _full_spec(expm.shape),
                  _full_spec(m0.shape), _full_spec(m1.shape), _full_spec(md.shape)],
        out_specs=out_specs,
        out_shape=out_shape,
    )(acc0, acc1, g0p, g1p, dp, bprev, expm, m0, m1, md)


def _tc_final(acc0, acc1, g4, d4, b4, batch2d, wih_t, bsum, wout_t, bout):
    def body(a0_r, a1_r, g_r, d_r, b4_r, bt_r, wih_r, bs_r, wo_r, bo_r, out_r,
             pool_r):
        pid = pl.program_id(0)
        acc = a0_r[...] + a1_r[...]
        g = g_r[...]
        H = g[:, :4]
        es = g[:, 4:5] + d_r[...][:, :1]
        es = jnp.where(es > 0, es, 0.2 * es)
        ws = jnp.exp(es)
        denom = acc[:, 4:5] + ws
        accH = acc[:, :4] + ws * H
        h = accH / denom + b4_r[...]
        h = jnp.where(h > 0, h, jnp.exp(h) - 1.0)
        bt = bt_r[...]
        oh = (bt == lax.broadcasted_iota(jnp.int32, (1, B), 1)).astype(f32)
        pooled = lax.dot_general(oh, h, (((0,), (0,)), ((), ())),
                                 preferred_element_type=f32)
        counts = jnp.sum(oh, axis=0)[:, None]

        @pl.when(pid == 0)
        def _():
            pool_r[...] = jnp.zeros_like(pool_r)

        pool_r[:, 0:4] = pool_r[:, 0:4] + pooled
        pool_r[:, 8:9] = pool_r[:, 8:9] + counts

        @pl.when(pid == GRID - 1)
        def _():
            pooled_m = pool_r[:, 0:4] / jnp.maximum(pool_r[:, 8:9], 1.0)
            gates = jnp.dot(pooled_m, wih_r[...],
                            preferred_element_type=f32) + bs_r[...]
            i_g = jax.nn.sigmoid(gates[:, 0:128])
            g_g = jnp.tanh(gates[:, 256:384])
            o_g = jax.nn.sigmoid(gates[:, 384:512])
            h_t = o_g * jnp.tanh(i_g * g_g)
            out_r[...] = jnp.dot(h_t, wo_r[...],
                                 preferred_element_type=f32) + bo_r[...]

    return pl.pallas_call(
        body,
        grid=(GRID,),
        in_specs=[_row_spec(16), _row_spec(16), _row_spec(16), _row_spec(16),
                  _full_spec(b4.shape), _row_spec(1), _full_spec(wih_t.shape),
                  _full_spec(bsum.shape), _full_spec(wout_t.shape),
                  _full_spec(bout.shape)],
        out_specs=pl.BlockSpec((B, 32), lambda i: (0, 0)),
        out_shape=jax.ShapeDtypeStruct((B, 32), f32),
        scratch_shapes=[pltpu.VMEM((B, 128), f32)],
    )(acc0, acc1, g4, d4, b4, batch2d, wih_t, bsum, wout_t, bout)


def _prep(W, a_s, a_d, heads, outc, half):
    Ws = (W.reshape(-1, heads, outc) * a_s[None]).sum(-1)
    Wd = (W.reshape(-1, heads, outc) * a_d[None]).sum(-1)
    m0 = jnp.concatenate([W[:, :half], Ws, Ws], axis=1)
    m1 = jnp.concatenate([W[:, half:], Ws, Ws], axis=1)
    md = jnp.concatenate([Wd, Wd], axis=1)
    return m0, m1, md


@jax.jit
def kernel(x, edge_index, batch, W1, as1, ad1, b1, W2, as2, ad2, b2,
           W3, as3, ad3, b3, W4, as4, ad4, b4, Wih, Whh, bih, bhh, Wout, bout):
    src = edge_index[0]
    dst = edge_index[1]

    m0_1, m1_1, md_1 = _prep(W1, as1, ad1, 8, 32, 128)
    m0_2, m1_2, md_2 = _prep(W2, as2, ad2, 8, 16, 64)
    m0_3, m1_3, md_3 = _prep(W3, as3, ad3, 8, 8, 32)
    Ws4 = (W4.reshape(-1, 1, 4) * as4[None]).sum(-1)
    Wd4 = (W4.reshape(-1, 1, 4) * ad4[None]).sum(-1)
    m4 = jnp.concatenate([W4] + [Ws4] * 12, axis=1)
    md4 = jnp.concatenate([Wd4] * 16, axis=1)

    exp1 = jnp.kron(jnp.eye(8, dtype=f32), jnp.ones((1, 32), f32))
    exp2 = jnp.kron(jnp.eye(8, dtype=f32), jnp.ones((1, 16), f32))
    exp3 = jnp.kron(jnp.eye(8, dtype=f32), jnp.ones((1, 8), f32))

    g0, g1, d = _tc_first(x, m0_1, m1_1, md_1)
    acc0, acc1 = _make_sc_edge(8, 32, 128, 144)(
        g0, g1, d, src, dst, jnp.zeros((N, 144), f32))

    g0, g1, d2 = _tc_boundary(acc0, acc1, g0, g1, d, b1.reshape(1, -1), exp1,
                              m0_2, m1_2, md_2, 128, True)
    acc0, acc1 = _make_sc_edge(8, 16, 64, 80)(
        g0, g1, d2, src, dst, jnp.zeros((N, 80), f32))

    g0, g1, d3 = _tc_boundary(acc0, acc1, g0, g1, d2, b2.reshape(1, -1), exp2,
                              m0_3, m1_3, md_3, 64, True)
    acc0, acc1 = _make_sc_edge(8, 8, 32, 48)(
        g0, g1, d3, src, dst, jnp.zeros((N, 48), f32))

    g4, d4 = _tc_boundary(acc0, acc1, g0, g1, d3, b3.reshape(1, -1), exp3,
                          m4, md4, md4, 32, False)
    acc0, acc1 = _make_sc_edge4()(g4, d4, src, dst, jnp.zeros((N, 16), f32))

    bsum = (bih + bhh).reshape(1, -1)
    return _tc_final(acc0, acc1, g4, d4, b4.reshape(1, -1),
                     batch.reshape(-1, 1), Wih.T, bsum, Wout.T, bout.reshape(1, -1))


# speedup vs baseline: 72.7827x; 1.9489x over previous
import functools
import jax
import jax.numpy as jnp
from jax import lax
from jax.experimental import pallas as pl
from jax.experimental.pallas import tpu as pltpu
from jax.experimental.pallas import tpu_sc as plsc

N = 10000
E = 160000
B = 16
NC = 2
NS = 16
f32 = jnp.float32


def _make_sc_edge(heads, outc, half, R):
    EPT = E // NS
    K = 80
    NBLK = EPT // K
    NCH = half // 16

    mesh = plsc.VectorSubcoreMesh(core_axis_name="c", subcore_axis_name="s")

    @functools.partial(
        pl.kernel,
        out_type=[jax.ShapeDtypeStruct((N, R), f32)] * 2,
        mesh=mesh,
        scratch_types=[
            pltpu.VMEM((K,), jnp.int32),
            pltpu.VMEM((K,), jnp.int32),
            pltpu.VMEM((K,), jnp.int32),
            pltpu.VMEM((K,), jnp.int32),
            pltpu.VMEM((K, R), f32),
            pltpu.VMEM((K, R), f32),
            pltpu.VMEM((K, 16), f32),
            pltpu.VMEM((K, 16), f32),
            pltpu.VMEM_SHARED((N, R), f32),
            pltpu.SemaphoreType.DMA,
            pltpu.SemaphoreType.DMA,
        ],
        compiler_params=pltpu.CompilerParams(use_tc_tiling_on_sc=False),
    )
    def sc_edge(g0, g1, d, src_h, dst_h, zz, acc0_o, acc1_o,
                idx_s0, idx_s1, idx_d0, idx_d1, gbuf0, gbuf1,
                dbuf0, dbuf1, acc_sh, sem0, sem1):
        c = lax.axis_index("c")
        s = lax.axis_index("s")

        def tile_rows(do):
            @pl.when(s < NS - 1)
            def _():
                do(pl.multiple_of(s * 632, 8), 632)

            @pl.when(s == NS - 1)
            def _():
                do((NS - 1) * 632, N - (NS - 1) * 632)

        tile_rows(lambda r0, nr: pltpu.sync_copy(
            zz.at[pl.ds(r0, nr)], acc_sh.at[pl.ds(r0, nr)]))
        plsc.subcore_barrier()

        mask8 = lax.iota(jnp.int32, 16) < 8

        idx_s = (idx_s0, idx_s1)
        idx_d = (idx_d0, idx_d1)
        gbuf = (gbuf0, gbuf1)
        dbuf = (dbuf0, dbuf1)
        sem = (sem0, sem1)

        def run(g_hbm, hb):
            base0 = s * EPT

            def prefetch(b, sl):
                base = pl.multiple_of(base0 + b * K, 8)
                pltpu.sync_copy(src_h.at[pl.ds(base, K)], idx_s[sl])
                pltpu.sync_copy(dst_h.at[pl.ds(base, K)], idx_d[sl])
                pltpu.async_copy(g_hbm.at[idx_s[sl]], gbuf[sl], sem[sl])
                pltpu.async_copy(d.at[idx_d[sl]], dbuf[sl], sem[sl])

            def wait_gather(sl):
                pltpu.make_async_copy(g_hbm.at[idx_s[sl]], gbuf[sl], sem[sl]).wait()
                pltpu.make_async_copy(d.at[idx_d[sl]], dbuf[sl], sem[sl]).wait()

            def compute(sl):
                gb = gbuf[sl]
                db = dbuf[sl]

                @plsc.parallel_loop(0, K, unroll=4)
                def _(e):
                    a = gb[e, pl.ds(half, 16)]
                    ee = a + db[e, :]
                    ee = jnp.where(ee > 0, ee, 0.2 * ee)
                    w = jnp.exp(ee)
                    gb[e, pl.ds(half, 16)] = w
                    for ch in range(NCH):
                        if outc >= 16:
                            wv = lax.broadcast(w[hb + (ch * 16) // outc], (16,))
                        else:
                            w_lo = lax.broadcast(w[hb + 2 * ch], (16,))
                            w_hi = lax.broadcast(w[hb + 2 * ch + 1], (16,))
                            wv = jnp.where(mask8, w_lo, w_hi)
                        gb[e, pl.ds(ch * 16, 16)] = gb[e, pl.ds(ch * 16, 16)] * wv

                pltpu.sync_copy(gb, acc_sh.at[idx_d[sl]], add=True)

            prefetch(0, 0)
            prefetch(1, 1)

            def pair(i, carry):
                b = 2 * i
                for sl in (0, 1):
                    wait_gather(sl)
                    compute(sl)
                    nb = b + 2 + sl

                    @pl.when(nb < NBLK)
                    def _():
                        prefetch(nb, sl)
                return carry

            lax.fori_loop(0, (NBLK - 1) // 2, pair, 0)
            wait_gather(0)
            compute(0)

        @pl.when(c == 0)
        def _():
            run(g0, 0)

        @pl.when(c == 1)
        def _():
            run(g1, heads // 2)

        plsc.subcore_barrier()

        @pl.when(c == 0)
        def _():
            tile_rows(lambda r0, nr: pltpu.sync_copy(
                acc_sh.at[pl.ds(r0, nr)], acc0_o.at[pl.ds(r0, nr)]))

        @pl.when(c == 1)
        def _():
            tile_rows(lambda r0, nr: pltpu.sync_copy(
                acc_sh.at[pl.ds(r0, nr)], acc1_o.at[pl.ds(r0, nr)]))

    return sc_edge


def _make_sc_edge4():
    EPT = E // (NS * NC)
    K = 40
    NBLK = EPT // K

    mesh = plsc.VectorSubcoreMesh(core_axis_name="c", subcore_axis_name="s")

    @functools.partial(
        pl.kernel,
        out_type=[jax.ShapeDtypeStruct((N, 16), f32)] * 2,
        mesh=mesh,
        scratch_types=[
            pltpu.VMEM((K,), jnp.int32),
            pltpu.VMEM((K,), jnp.int32),
            pltpu.VMEM((K,), jnp.int32),
            pltpu.VMEM((K,), jnp.int32),
            pltpu.VMEM((K, 16), f32),
            pltpu.VMEM((K, 16), f32),
            pltpu.VMEM((K, 16), f32),
            pltpu.VMEM((K, 16), f32),
            pltpu.VMEM_SHARED((N, 16), f32),
            pltpu.SemaphoreType.DMA,
            pltpu.SemaphoreType.DMA,
        ],
        compiler_params=pltpu.CompilerParams(use_tc_tiling_on_sc=False),
    )
    def sc_edge4(g, d, src_h, dst_h, zz, acc0_o, acc1_o,
                 idx_s0, idx_s1, idx_d0, idx_d1, gbuf0, gbuf1,
                 dbuf0, dbuf1, acc_sh, sem0, sem1):
        c = lax.axis_index("c")
        s = lax.axis_index("s")

        def tile_rows(do):
            @pl.when(s < NS - 1)
            def _():
                do(pl.multiple_of(s * 632, 8), 632)

            @pl.when(s == NS - 1)
            def _():
                do((NS - 1) * 632, N - (NS - 1) * 632)

        tile_rows(lambda r0, nr: pltpu.sync_copy(
            zz.at[pl.ds(r0, nr)], acc_sh.at[pl.ds(r0, nr)]))
        plsc.subcore_barrier()

        mask4 = lax.iota(jnp.int32, 16) < 4
        wid = s * NC + c
        base0 = wid * EPT

        idx_s = (idx_s0, idx_s1)
        idx_d = (idx_d0, idx_d1)
        gbuf = (gbuf0, gbuf1)
        dbuf = (dbuf0, dbuf1)
        sem = (sem0, sem1)

        def prefetch(b, sl):
            base = pl.multiple_of(base0 + b * K, 8)
            pltpu.sync_copy(src_h.at[pl.ds(base, K)], idx_s[sl])
            pltpu.sync_copy(dst_h.at[pl.ds(base, K)], idx_d[sl])
            pltpu.async_copy(g.at[idx_s[sl]], gbuf[sl], sem[sl])
            pltpu.async_copy(d.at[idx_d[sl]], dbuf[sl], sem[sl])

        def wait_gather(sl):
            pltpu.make_async_copy(g.at[idx_s[sl]], gbuf[sl], sem[sl]).wait()
            pltpu.make_async_copy(d.at[idx_d[sl]], dbuf[sl], sem[sl]).wait()

        def compute(sl):
            gb = gbuf[sl]
            db = dbuf[sl]

            @plsc.parallel_loop(0, K, unroll=4)
            def _(e):
                a_row = gb[e, :]
                ev = a_row + db[e, :]
                es = lax.broadcast(ev[4], (16,))
                es = jnp.where(es > 0, es, 0.2 * es)
                w = jnp.exp(es)
                gb[e, :] = jnp.where(mask4, a_row * w, w)

            pltpu.sync_copy(gb, acc_sh.at[idx_d[sl]], add=True)

        prefetch(0, 0)
        prefetch(1, 1)

        def pair(i, carry):
            b = 2 * i
            for sl in (0, 1):
                wait_gather(sl)
                compute(sl)
                nb = b + 2 + sl

                @pl.when(nb < NBLK)
                def _():
                    prefetch(nb, sl)
            return carry

        lax.fori_loop(0, (NBLK - 1) // 2, pair, 0)
        wait_gather(0)
        compute(0)
        plsc.subcore_barrier()

        @pl.when(c == 0)
        def _():
            tile_rows(lambda r0, nr: pltpu.sync_copy(
                acc_sh.at[pl.ds(r0, nr)], acc0_o.at[pl.ds(r0, nr)]))

        @pl.when(c == 1)
        def _():
            tile_rows(lambda r0, nr: pltpu.sync_copy(
                acc_sh.at[pl.ds(r0, nr)], acc1_o.at[pl.ds(r0, nr)]))

    return sc_edge4


BLK = 2000
GRID = N // BLK


def _row_spec(ncol):
    return pl.BlockSpec((BLK, ncol), lambda i: (i, 0))


def _full_spec(shape):
    return pl.BlockSpec(shape, lambda i: (0,) * len(shape))


def _tc_first(x, m0, m1, md):
    def body(x_r, m0_r, m1_r, md_r, g0_r, g1_r, d_r):
        xx = x_r[...]
        g0_r[...] = jnp.dot(xx, m0_r[...], preferred_element_type=f32)
        g1_r[...] = jnp.dot(xx, m1_r[...], preferred_element_type=f32)
        d_r[...] = jnp.dot(xx, md_r[...], preferred_element_type=f32)

    R = m0.shape[1]
    return pl.pallas_call(
        body,
        grid=(GRID,),
        in_specs=[_row_spec(x.shape[1]), _full_spec(m0.shape),
                  _full_spec(m1.shape), _full_spec(md.shape)],
        out_specs=[_row_spec(R), _row_spec(R), _row_spec(16)],
        out_shape=[
            jax.ShapeDtypeStruct((N, R), f32),
            jax.ShapeDtypeStruct((N, R), f32),
            jax.ShapeDtypeStruct((N, 16), f32),
        ],
    )(x, m0, m1, md)


def _combine(acc0, acc1, g0p, g1p, dp, bprev, expm, half):
    H = jnp.concatenate([g0p[:, :half], g1p[:, :half]], axis=1)
    asrc = g0p[:, half:half + 8]
    adst = dp[:, :8]
    es = asrc + adst
    es = jnp.where(es > 0, es, 0.2 * es)
    ws = jnp.exp(es)
    denom = acc0[:, half:half + 8] + ws
    wsE = jnp.dot(ws, expm, preferred_element_type=f32)
    denE = jnp.dot(denom, expm, preferred_element_type=f32)
    accH = jnp.concatenate([acc0[:, :half], acc1[:, :half]], axis=1) + wsE * H
    h = accH / denE + bprev
    return jnp.where(h > 0, h, jnp.exp(h) - 1.0)


def _tc_boundary(acc0, acc1, g0p, g1p, dp, bprev, expm, m0, m1, md, half, two_out):
    def body(a0_r, a1_r, g0_r, g1_r, d_r, b_r, e_r, m0_r, m1_r, md_r, *outs):
        h = _combine(a0_r[...], a1_r[...], g0_r[...], g1_r[...], d_r[...],
                     b_r[...], e_r[...], half)
        outs[0][...] = jnp.dot(h, m0_r[...], preferred_element_type=f32)
        if two_out:
            outs[1][...] = jnp.dot(h, m1_r[...], preferred_element_type=f32)
            outs[2][...] = jnp.dot(h, md_r[...], preferred_element_type=f32)
        else:
            outs[1][...] = jnp.dot(h, md_r[...], preferred_element_type=f32)

    R = m0.shape[1]
    if two_out:
        out_shape = [
            jax.ShapeDtypeStruct((N, R), f32),
            jax.ShapeDtypeStruct((N, R), f32),
            jax.ShapeDtypeStruct((N, 16), f32),
        ]
        out_specs = [_row_spec(R), _row_spec(R), _row_spec(16)]
    else:
        out_shape = [
            jax.ShapeDtypeStruct((N, R), f32),
            jax.ShapeDtypeStruct((N, 16), f32),
        ]
        out_specs = [_row_spec(R), _row_spec(16)]
    Rp = acc0.shape[1]
    return pl.pallas_call(
        body,
        grid=(GRID,),
        in_specs=[_row_spec(Rp), _row_spec(Rp), _row_spec(Rp), _row_spec(Rp),
                  _row_spec(16), _full_spec(bprev.shape), _full_spec(expm.shape),
                  _full_spec(m0.shape), _full_spec(m1.shape), _full_spec(md.shape)],
        out_specs=out_specs,
        out_shape=out_shape,
    )(acc0, acc1, g0p, g1p, dp, bprev, expm, m0, m1, md)


def _tc_final(acc0, acc1, g4, d4, b4, batch2d, wih_t, bsum, wout_t, bout):
    def body(a0_r, a1_r, g_r, d_r, b4_r, bt_r, wih_r, bs_r, wo_r, bo_r, out_r,
             pool_r):
        pid = pl.program_id(0)
        acc = a0_r[...] + a1_r[...]
        g = g_r[...]
        H = g[:, :4]
        es = g[:, 4:5] + d_r[...][:, :1]
        es = jnp.where(es > 0, es, 0.2 * es)
        ws = jnp.exp(es)
        denom = acc[:, 4:5] + ws
        accH = acc[:, :4] + ws * H
        h = accH / denom + b4_r[...]
        h = jnp.where(h > 0, h, jnp.exp(h) - 1.0)
        bt = bt_r[...]
        oh = (bt == lax.broadcasted_iota(jnp.int32, (1, B), 1)).astype(f32)
        pooled = lax.dot_general(oh, h, (((0,), (0,)), ((), ())),
                                 preferred_element_type=f32)
        counts = jnp.sum(oh, axis=0)[:, None]

        @pl.when(pid == 0)
        def _():
            pool_r[...] = jnp.zeros_like(pool_r)

        pool_r[:, 0:4] = pool_r[:, 0:4] + pooled
        pool_r[:, 8:9] = pool_r[:, 8:9] + counts

        @pl.when(pid == GRID - 1)
        def _():
            pooled_m = pool_r[:, 0:4] / jnp.maximum(pool_r[:, 8:9], 1.0)
            gates = jnp.dot(pooled_m, wih_r[...],
                            preferred_element_type=f32) + bs_r[...]
            i_g = jax.nn.sigmoid(gates[:, 0:128])
            g_g = jnp.tanh(gates[:, 256:384])
            o_g = jax.nn.sigmoid(gates[:, 384:512])
            h_t = o_g * jnp.tanh(i_g * g_g)
            out_r[...] = jnp.dot(h_t, wo_r[...],
                                 preferred_element_type=f32) + bo_r[...]

    return pl.pallas_call(
        body,
        grid=(GRID,),
        in_specs=[_row_spec(16), _row_spec(16), _row_spec(16), _row_spec(16),
                  _full_spec(b4.shape), _row_spec(1), _full_spec(wih_t.shape),
                  _full_spec(bsum.shape), _full_spec(wout_t.shape),
                  _full_spec(bout.shape)],
        out_specs=pl.BlockSpec((B, 32), lambda i: (0, 0)),
        out_shape=jax.ShapeDtypeStruct((B, 32), f32),
        scratch_shapes=[pltpu.VMEM((B, 128), f32)],
    )(acc0, acc1, g4, d4, b4, batch2d, wih_t, bsum, wout_t, bout)


def _prep(W, a_s, a_d, heads, outc, half):
    Ws = (W.reshape(-1, heads, outc) * a_s[None]).sum(-1)
    Wd = (W.reshape(-1, heads, outc) * a_d[None]).sum(-1)
    m0 = jnp.concatenate([W[:, :half], Ws, Ws], axis=1)
    m1 = jnp.concatenate([W[:, half:], Ws, Ws], axis=1)
    md = jnp.concatenate([Wd, Wd], axis=1)
    return m0, m1, md


@jax.jit
def kernel(x, edge_index, batch, W1, as1, ad1, b1, W2, as2, ad2, b2,
           W3, as3, ad3, b3, W4, as4, ad4, b4, Wih, Whh, bih, bhh, Wout, bout):
    src = edge_index[0]
    dst = edge_index[1]

    m0_1, m1_1, md_1 = _prep(W1, as1, ad1, 8, 32, 128)
    m0_2, m1_2, md_2 = _prep(W2, as2, ad2, 8, 16, 64)
    m0_3, m1_3, md_3 = _prep(W3, as3, ad3, 8, 8, 32)
    Ws4 = (W4.reshape(-1, 1, 4) * as4[None]).sum(-1)
    Wd4 = (W4.reshape(-1, 1, 4) * ad4[None]).sum(-1)
    m4 = jnp.concatenate([W4] + [Ws4] * 12, axis=1)
    md4 = jnp.concatenate([Wd4] * 16, axis=1)

    exp1 = jnp.kron(jnp.eye(8, dtype=f32), jnp.ones((1, 32), f32))
    exp2 = jnp.kron(jnp.eye(8, dtype=f32), jnp.ones((1, 16), f32))
    exp3 = jnp.kron(jnp.eye(8, dtype=f32), jnp.ones((1, 8), f32))

    g0, g1, d = _tc_first(x, m0_1, m1_1, md_1)
    acc0, acc1 = _make_sc_edge(8, 32, 128, 144)(
        g0, g1, d, src, dst, jnp.zeros((N, 144), f32))

    g0, g1, d2 = _tc_boundary(acc0, acc1, g0, g1, d, b1.reshape(1, -1), exp1,
                              m0_2, m1_2, md_2, 128, True)
    acc0, acc1 = _make_sc_edge(8, 16, 64, 80)(
        g0, g1, d2, src, dst, jnp.zeros((N, 80), f32))

    g0, g1, d3 = _tc_boundary(acc0, acc1, g0, g1, d2, b2.reshape(1, -1), exp2,
                              m0_3, m1_3, md_3, 64, True)
    acc0, acc1 = _make_sc_edge(8, 8, 32, 48)(
        g0, g1, d3, src, dst, jnp.zeros((N, 48), f32))

    g4, d4 = _tc_boundary(acc0, acc1, g0, g1, d3, b3.reshape(1, -1), exp3,
                          m4, md4, md4, 32, False)
    acc0, acc1 = _make_sc_edge4()(g4, d4, src, dst, jnp.zeros((N, 16), f32))

    bsum = (bih + bhh).reshape(1, -1)
    return _tc_final(acc0, acc1, g4, d4, b4.reshape(1, -1),
                     batch.reshape(-1, 1), Wih.T, bsum, Wout.T, bout.reshape(1, -1))


# speedup vs baseline: 88.5566x; 1.2167x over previous
import functools
import jax
import jax.numpy as jnp
from jax import lax
from jax.experimental import pallas as pl
from jax.experimental.pallas import tpu as pltpu
from jax.experimental.pallas import tpu_sc as plsc

N = 10000
E = 160000
B = 16
NC = 2
NS = 16
f32 = jnp.float32


def _make_sc_edge(heads, outc, half, R, K):
    EPT = E // NS
    NBLK = EPT // K
    KT = EPT - NBLK * K
    NCH = half // 16

    mesh = plsc.VectorSubcoreMesh(core_axis_name="c", subcore_axis_name="s")

    @functools.partial(
        pl.kernel,
        out_type=[jax.ShapeDtypeStruct((N, R), f32)] * 2,
        mesh=mesh,
        scratch_types=[
            pltpu.VMEM((K,), jnp.int32),
            pltpu.VMEM((K,), jnp.int32),
            pltpu.VMEM((K,), jnp.int32),
            pltpu.VMEM((K,), jnp.int32),
            pltpu.VMEM((K, R), f32),
            pltpu.VMEM((K, R), f32),
            pltpu.VMEM((K, 16), f32),
            pltpu.VMEM((K, 16), f32),
            pltpu.VMEM((KT,), jnp.int32),
            pltpu.VMEM((KT,), jnp.int32),
            pltpu.VMEM((KT, R), f32),
            pltpu.VMEM((KT, 16), f32),
            pltpu.VMEM_SHARED((N, R), f32),
            pltpu.SemaphoreType.DMA,
            pltpu.SemaphoreType.DMA,
            pltpu.SemaphoreType.DMA,
        ],
        compiler_params=pltpu.CompilerParams(use_tc_tiling_on_sc=False),
    )
    def sc_edge(g0, g1, d, src_h, dst_h, zz, acc0_o, acc1_o,
                idx_s0, idx_s1, idx_d0, idx_d1, gbuf0, gbuf1,
                dbuf0, dbuf1, idx_st, idx_dt, gbuf_t, dbuf_t,
                acc_sh, sem0, sem1, sem_t):
        c = lax.axis_index("c")
        s = lax.axis_index("s")

        def tile_rows(do):
            @pl.when(s < NS - 1)
            def _():
                do(pl.multiple_of(s * 632, 8), 632)

            @pl.when(s == NS - 1)
            def _():
                do((NS - 1) * 632, N - (NS - 1) * 632)

        tile_rows(lambda r0, nr: pltpu.sync_copy(
            zz.at[pl.ds(r0, nr)], acc_sh.at[pl.ds(r0, nr)]))
        plsc.subcore_barrier()

        mask8 = lax.iota(jnp.int32, 16) < 8

        idx_s = (idx_s0, idx_s1)
        idx_d = (idx_d0, idx_d1)
        gbuf = (gbuf0, gbuf1)
        dbuf = (dbuf0, dbuf1)
        sem = (sem0, sem1)

        def run(g_hbm, hb):
            base0 = s * EPT

            def prefetch(b, sl):
                base = pl.multiple_of(base0 + b * K, 8)
                pltpu.sync_copy(src_h.at[pl.ds(base, K)], idx_s[sl])
                pltpu.sync_copy(dst_h.at[pl.ds(base, K)], idx_d[sl])
                pltpu.async_copy(g_hbm.at[idx_s[sl]], gbuf[sl], sem[sl])
                pltpu.async_copy(d.at[idx_d[sl]], dbuf[sl], sem[sl])

            def wait_gather(sl):
                pltpu.make_async_copy(g_hbm.at[idx_s[sl]], gbuf[sl], sem[sl]).wait()
                pltpu.make_async_copy(d.at[idx_d[sl]], dbuf[sl], sem[sl]).wait()

            def edge_body(gb, db):
                def body(e):
                    a = gb[e, pl.ds(half, 16)]
                    ee = a + db[e, :]
                    ee = jnp.where(ee > 0, ee, 0.2 * ee)
                    w = jnp.exp(ee)
                    gb[e, pl.ds(half, 16)] = w
                    for ch in range(NCH):
                        if outc >= 16:
                            wv = lax.broadcast(w[hb + (ch * 16) // outc], (16,))
                        else:
                            w_lo = lax.broadcast(w[hb + 2 * ch], (16,))
                            w_hi = lax.broadcast(w[hb + 2 * ch + 1], (16,))
                            wv = jnp.where(mask8, w_lo, w_hi)
                        gb[e, pl.ds(ch * 16, 16)] = gb[e, pl.ds(ch * 16, 16)] * wv
                return body

            def compute(sl):
                gb = gbuf[sl]
                plsc.parallel_loop(0, K, unroll=4)(edge_body(gb, dbuf[sl]))
                pltpu.sync_copy(gb, acc_sh.at[idx_d[sl]], add=True)

            baset = pl.multiple_of(base0 + NBLK * K, 8)
            pltpu.sync_copy(src_h.at[pl.ds(baset, KT)], idx_st)
            pltpu.sync_copy(dst_h.at[pl.ds(baset, KT)], idx_dt)
            pltpu.async_copy(g_hbm.at[idx_st], gbuf_t, sem_t)
            pltpu.async_copy(d.at[idx_dt], dbuf_t, sem_t)

            prefetch(0, 0)
            prefetch(1, 1)

            def pair(i, carry):
                b = 2 * i
                for sl in (0, 1):
                    wait_gather(sl)
                    compute(sl)
                    nb = b + 2 + sl

                    @pl.when(nb < NBLK)
                    def _():
                        prefetch(nb, sl)
                return carry

            lax.fori_loop(0, NBLK // 2, pair, 0)
            if NBLK % 2:
                wait_gather(0)
                compute(0)
            pltpu.make_async_copy(g_hbm.at[idx_st], gbuf_t, sem_t).wait()
            pltpu.make_async_copy(d.at[idx_dt], dbuf_t, sem_t).wait()
            plsc.parallel_loop(0, KT, unroll=2)(edge_body(gbuf_t, dbuf_t))
            pltpu.sync_copy(gbuf_t, acc_sh.at[idx_dt], add=True)

        @pl.when(c == 0)
        def _():
            run(g0, 0)

        @pl.when(c == 1)
        def _():
            run(g1, heads // 2)

        plsc.subcore_barrier()

        @pl.when(c == 0)
        def _():
            tile_rows(lambda r0, nr: pltpu.sync_copy(
                acc_sh.at[pl.ds(r0, nr)], acc0_o.at[pl.ds(r0, nr)]))

        @pl.when(c == 1)
        def _():
            tile_rows(lambda r0, nr: pltpu.sync_copy(
                acc_sh.at[pl.ds(r0, nr)], acc1_o.at[pl.ds(r0, nr)]))

    return sc_edge


def _make_sc_edge4():
    EPT = E // (NS * NC)
    K = 128
    NBLK = EPT // K
    KT = EPT - NBLK * K

    mesh = plsc.VectorSubcoreMesh(core_axis_name="c", subcore_axis_name="s")

    @functools.partial(
        pl.kernel,
        out_type=[jax.ShapeDtypeStruct((N, 16), f32)] * 2,
        mesh=mesh,
        scratch_types=[
            pltpu.VMEM((K,), jnp.int32),
            pltpu.VMEM((K,), jnp.int32),
            pltpu.VMEM((K,), jnp.int32),
            pltpu.VMEM((K,), jnp.int32),
            pltpu.VMEM((K, 16), f32),
            pltpu.VMEM((K, 16), f32),
            pltpu.VMEM((K, 16), f32),
            pltpu.VMEM((K, 16), f32),
            pltpu.VMEM((KT,), jnp.int32),
            pltpu.VMEM((KT,), jnp.int32),
            pltpu.VMEM((KT, 16), f32),
            pltpu.VMEM((KT, 16), f32),
            pltpu.VMEM_SHARED((N, 16), f32),
            pltpu.SemaphoreType.DMA,
            pltpu.SemaphoreType.DMA,
            pltpu.SemaphoreType.DMA,
        ],
        compiler_params=pltpu.CompilerParams(use_tc_tiling_on_sc=False),
    )
    def sc_edge4(g, d, src_h, dst_h, zz, acc0_o, acc1_o,
                 idx_s0, idx_s1, idx_d0, idx_d1, gbuf0, gbuf1,
                 dbuf0, dbuf1, idx_st, idx_dt, gbuf_t, dbuf_t,
                 acc_sh, sem0, sem1, sem_t):
        c = lax.axis_index("c")
        s = lax.axis_index("s")

        def tile_rows(do):
            @pl.when(s < NS - 1)
            def _():
                do(pl.multiple_of(s * 632, 8), 632)

            @pl.when(s == NS - 1)
            def _():
                do((NS - 1) * 632, N - (NS - 1) * 632)

        tile_rows(lambda r0, nr: pltpu.sync_copy(
            zz.at[pl.ds(r0, nr)], acc_sh.at[pl.ds(r0, nr)]))
        plsc.subcore_barrier()

        mask4 = lax.iota(jnp.int32, 16) < 4
        wid = s * NC + c
        base0 = wid * EPT

        idx_s = (idx_s0, idx_s1)
        idx_d = (idx_d0, idx_d1)
        gbuf = (gbuf0, gbuf1)
        dbuf = (dbuf0, dbuf1)
        sem = (sem0, sem1)

        def prefetch(b, sl):
            base = pl.multiple_of(base0 + b * K, 8)
            pltpu.sync_copy(src_h.at[pl.ds(base, K)], idx_s[sl])
            pltpu.sync_copy(dst_h.at[pl.ds(base, K)], idx_d[sl])
            pltpu.async_copy(g.at[idx_s[sl]], gbuf[sl], sem[sl])
            pltpu.async_copy(d.at[idx_d[sl]], dbuf[sl], sem[sl])

        def wait_gather(sl):
            pltpu.make_async_copy(g.at[idx_s[sl]], gbuf[sl], sem[sl]).wait()
            pltpu.make_async_copy(d.at[idx_d[sl]], dbuf[sl], sem[sl]).wait()

        def edge_body(gb, db):
            def body(e):
                a_row = gb[e, :]
                ev = a_row + db[e, :]
                es = lax.broadcast(ev[4], (16,))
                es = jnp.where(es > 0, es, 0.2 * es)
                w = jnp.exp(es)
                gb[e, :] = jnp.where(mask4, a_row * w, w)
            return body

        def compute(sl):
            gb = gbuf[sl]
            plsc.parallel_loop(0, K, unroll=4)(edge_body(gb, dbuf[sl]))
            pltpu.sync_copy(gb, acc_sh.at[idx_d[sl]], add=True)

        baset = pl.multiple_of(base0 + NBLK * K, 8)
        pltpu.sync_copy(src_h.at[pl.ds(baset, KT)], idx_st)
        pltpu.sync_copy(dst_h.at[pl.ds(baset, KT)], idx_dt)
        pltpu.async_copy(g.at[idx_st], gbuf_t, sem_t)
        pltpu.async_copy(d.at[idx_dt], dbuf_t, sem_t)

        prefetch(0, 0)
        prefetch(1, 1)

        def pair(i, carry):
            b = 2 * i
            for sl in (0, 1):
                wait_gather(sl)
                compute(sl)
                nb = b + 2 + sl

                @pl.when(nb < NBLK)
                def _():
                    prefetch(nb, sl)
            return carry

        lax.fori_loop(0, NBLK // 2, pair, 0)
        if NBLK % 2:
            wait_gather(0)
            compute(0)
        pltpu.make_async_copy(g.at[idx_st], gbuf_t, sem_t).wait()
        pltpu.make_async_copy(d.at[idx_dt], dbuf_t, sem_t).wait()
        plsc.parallel_loop(0, KT, unroll=2)(edge_body(gbuf_t, dbuf_t))
        pltpu.sync_copy(gbuf_t, acc_sh.at[idx_dt], add=True)
        plsc.subcore_barrier()

        @pl.when(c == 0)
        def _():
            tile_rows(lambda r0, nr: pltpu.sync_copy(
                acc_sh.at[pl.ds(r0, nr)], acc0_o.at[pl.ds(r0, nr)]))

        @pl.when(c == 1)
        def _():
            tile_rows(lambda r0, nr: pltpu.sync_copy(
                acc_sh.at[pl.ds(r0, nr)], acc1_o.at[pl.ds(r0, nr)]))

    return sc_edge4


BLK = 2000
GRID = N // BLK


def _row_spec(ncol):
    return pl.BlockSpec((BLK, ncol), lambda i: (i, 0))


def _full_spec(shape):
    return pl.BlockSpec(shape, lambda i: (0,) * len(shape))


def _tc_first(x, m0, m1, md):
    def body(x_r, m0_r, m1_r, md_r, g0_r, g1_r, d_r):
        xx = x_r[...]
        g0_r[...] = jnp.dot(xx, m0_r[...], preferred_element_type=f32)
        g1_r[...] = jnp.dot(xx, m1_r[...], preferred_element_type=f32)
        d_r[...] = jnp.dot(xx, md_r[...], preferred_element_type=f32)

    R = m0.shape[1]
    return pl.pallas_call(
        body,
        grid=(GRID,),
        in_specs=[_row_spec(x.shape[1]), _full_spec(m0.shape),
                  _full_spec(m1.shape), _full_spec(md.shape)],
        out_specs=[_row_spec(R), _row_spec(R), _row_spec(16)],
        out_shape=[
            jax.ShapeDtypeStruct((N, R), f32),
            jax.ShapeDtypeStruct((N, R), f32),
            jax.ShapeDtypeStruct((N, 16), f32),
        ],
    )(x, m0, m1, md)


def _combine(acc0, acc1, g0p, g1p, dp, bprev, expm, half):
    H = jnp.concatenate([g0p[:, :half], g1p[:, :half]], axis=1)
    asrc = g0p[:, half:half + 8]
    adst = dp[:, :8]
    es = asrc + adst
    es = jnp.where(es > 0, es, 0.2 * es)
    ws = jnp.exp(es)
    denom = acc0[:, half:half + 8] + ws
    wsE = jnp.dot(ws, expm, preferred_element_type=f32)
    denE = jnp.dot(denom, expm, preferred_element_type=f32)
    accH = jnp.concatenate([acc0[:, :half], acc1[:, :half]], axis=1) + wsE * H
    h = accH / denE + bprev
    return jnp.where(h > 0, h, jnp.exp(h) - 1.0)


def _tc_boundary(acc0, acc1, g0p, g1p, dp, bprev, expm, m0, m1, md, half, two_out):
    def body(a0_r, a1_r, g0_r, g1_r, d_r, b_r, e_r, m0_r, m1_r, md_r, *outs):
        h = _combine(a0_r[...], a1_r[...], g0_r[...], g1_r[...], d_r[...],
                     b_r[...], e_r[...], half)
        outs[0][...] = jnp.dot(h, m0_r[...], preferred_element_type=f32)
        if two_out:
            outs[1][...] = jnp.dot(h, m1_r[...], preferred_element_type=f32)
            outs[2][...] = jnp.dot(h, md_r[...], preferred_element_type=f32)
        else:
            outs[1][...] = jnp.dot(h, md_r[...], preferred_element_type=f32)

    R = m0.shape[1]
    if two_out:
        out_shape = [
            jax.ShapeDtypeStruct((N, R), f32),
            jax.ShapeDtypeStruct((N, R), f32),
            jax.ShapeDtypeStruct((N, 16), f32),
        ]
        out_specs = [_row_spec(R), _row_spec(R), _row_spec(16)]
    else:
        out_shape = [
            jax.ShapeDtypeStruct((N, R), f32),
            jax.ShapeDtypeStruct((N, 16), f32),
        ]
        out_specs = [_row_spec(R), _row_spec(16)]
    Rp = acc0.shape[1]
    return pl.pallas_call(
        body,
        grid=(GRID,),
        in_specs=[_row_spec(Rp), _row_spec(Rp), _row_spec(Rp), _row_spec(Rp),
                  _row_spec(16), _full_spec(bprev.shape), _full_spec(expm.shape),
                  _full_spec(m0.shape), _full_spec(m1.shape), _full_spec(md.shape)],
        out_specs=out_specs,
        out_shape=out_shape,
    )(acc0, acc1, g0p, g1p, dp, bprev, expm, m0, m1, md)


def _tc_final(acc0, acc1, g4, d4, b4, batch2d, wih_t, bsum, wout_t, bout):
    def body(a0_r, a1_r, g_r, d_r, b4_r, bt_r, wih_r, bs_r, wo_r, bo_r, out_r,
             pool_r):
        pid = pl.program_id(0)
        acc = a0_r[...] + a1_r[...]
        g = g_r[...]
        H = g[:, :4]
        es = g[:, 4:5] + d_r[...][:, :1]
        es = jnp.where(es > 0, es, 0.2 * es)
        ws = jnp.exp(es)
        denom = acc[:, 4:5] + ws
        accH = acc[:, :4] + ws * H
        h = accH / denom + b4_r[...]
        h = jnp.where(h > 0, h, jnp.exp(h) - 1.0)
        bt = bt_r[...]
        oh = (bt == lax.broadcasted_iota(jnp.int32, (1, B), 1)).astype(f32)
        pooled = lax.dot_general(oh, h, (((0,), (0,)), ((), ())),
                                 preferred_element_type=f32)
        counts = jnp.sum(oh, axis=0)[:, None]

        @pl.when(pid == 0)
        def _():
            pool_r[...] = jnp.zeros_like(pool_r)

        pool_r[:, 0:4] = pool_r[:, 0:4] + pooled
        pool_r[:, 8:9] = pool_r[:, 8:9] + counts

        @pl.when(pid == GRID - 1)
        def _():
            pooled_m = pool_r[:, 0:4] / jnp.maximum(pool_r[:, 8:9], 1.0)
            gates = jnp.dot(pooled_m, wih_r[...],
                            preferred_element_type=f32) + bs_r[...]
            i_g = jax.nn.sigmoid(gates[:, 0:128])
            g_g = jnp.tanh(gates[:, 256:384])
            o_g = jax.nn.sigmoid(gates[:, 384:512])
            h_t = o_g * jnp.tanh(i_g * g_g)
            out_r[...] = jnp.dot(h_t, wo_r[...],
                                 preferred_element_type=f32) + bo_r[...]

    return pl.pallas_call(
        body,
        grid=(GRID,),
        in_specs=[_row_spec(16), _row_spec(16), _row_spec(16), _row_spec(16),
                  _full_spec(b4.shape), _row_spec(1), _full_spec(wih_t.shape),
                  _full_spec(bsum.shape), _full_spec(wout_t.shape),
                  _full_spec(bout.shape)],
        out_specs=pl.BlockSpec((B, 32), lambda i: (0, 0)),
        out_shape=jax.ShapeDtypeStruct((B, 32), f32),
        scratch_shapes=[pltpu.VMEM((B, 128), f32)],
    )(acc0, acc1, g4, d4, b4, batch2d, wih_t, bsum, wout_t, bout)


def _prep(W, a_s, a_d, heads, outc, half):
    Ws = (W.reshape(-1, heads, outc) * a_s[None]).sum(-1)
    Wd = (W.reshape(-1, heads, outc) * a_d[None]).sum(-1)
    m0 = jnp.concatenate([W[:, :half], Ws, Ws], axis=1)
    m1 = jnp.concatenate([W[:, half:], Ws, Ws], axis=1)
    md = jnp.concatenate([Wd, Wd], axis=1)
    return m0, m1, md


@jax.jit
def kernel(x, edge_index, batch, W1, as1, ad1, b1, W2, as2, ad2, b2,
           W3, as3, ad3, b3, W4, as4, ad4, b4, Wih, Whh, bih, bhh, Wout, bout):
    src = edge_index[0]
    dst = edge_index[1]

    m0_1, m1_1, md_1 = _prep(W1, as1, ad1, 8, 32, 128)
    m0_2, m1_2, md_2 = _prep(W2, as2, ad2, 8, 16, 64)
    m0_3, m1_3, md_3 = _prep(W3, as3, ad3, 8, 8, 32)
    Ws4 = (W4.reshape(-1, 1, 4) * as4[None]).sum(-1)
    Wd4 = (W4.reshape(-1, 1, 4) * ad4[None]).sum(-1)
    m4 = jnp.concatenate([W4] + [Ws4] * 12, axis=1)
    md4 = jnp.concatenate([Wd4] * 16, axis=1)

    exp1 = jnp.kron(jnp.eye(8, dtype=f32), jnp.ones((1, 32), f32))
    exp2 = jnp.kron(jnp.eye(8, dtype=f32), jnp.ones((1, 16), f32))
    exp3 = jnp.kron(jnp.eye(8, dtype=f32), jnp.ones((1, 8), f32))

    g0, g1, d = _tc_first(x, m0_1, m1_1, md_1)
    acc0, acc1 = _make_sc_edge(8, 32, 128, 144, 104)(
        g0, g1, d, src, dst, jnp.zeros((N, 144), f32))

    g0, g1, d2 = _tc_boundary(acc0, acc1, g0, g1, d, b1.reshape(1, -1), exp1,
                              m0_2, m1_2, md_2, 128, True)
    acc0, acc1 = _make_sc_edge(8, 16, 64, 80, 128)(
        g0, g1, d2, src, dst, jnp.zeros((N, 80), f32))

    g0, g1, d3 = _tc_boundary(acc0, acc1, g0, g1, d2, b2.reshape(1, -1), exp2,
                              m0_3, m1_3, md_3, 64, True)
    acc0, acc1 = _make_sc_edge(8, 8, 32, 48, 128)(
        g0, g1, d3, src, dst, jnp.zeros((N, 48), f32))

    g4, d4 = _tc_boundary(acc0, acc1, g0, g1, d3, b3.reshape(1, -1), exp3,
                          m4, md4, md4, 32, False)
    acc0, acc1 = _make_sc_edge4()(g4, d4, src, dst, jnp.zeros((N, 16), f32))

    bsum = (bih + bhh).reshape(1, -1)
    return _tc_final(acc0, acc1, g4, d4, b4.reshape(1, -1),
                     batch.reshape(-1, 1), Wih.T, bsum, Wout.T, bout.reshape(1, -1))
